# baseline (speedup 1.0000x reference)
import jax, jax.numpy as jnp
from jax.experimental import pallas as pl

NUM_GRAPHS = 64


def _gatv2(x, src, dst, edge_attr, Wl, bl, Wr, br, We, att, bias, H, C):
    n = x.shape[0]
    xl = (x @ Wl + bl).reshape(n, H, C)
    xr = (x @ Wr + br).reshape(n, H, C)
    ef = (edge_attr @ We).reshape(-1, H, C)
    m = xl[src] + xr[dst] + ef
    m = jnp.where(m > 0, m, 0.2 * m)
    logits = jnp.sum(m * att[None, :, :], axis=-1)
    lmax = jax.ops.segment_max(logits, dst, num_segments=n)
    lmax = jnp.where(jnp.isfinite(lmax), lmax, 0.0)
    ex = jnp.exp(logits - lmax[dst])
    denom = jax.ops.segment_sum(ex, dst, num_segments=n)
    alpha = ex / (denom[dst] + 1e-16)
    msg = xl[src] * alpha[..., None]
    out = jax.ops.segment_sum(msg, dst, num_segments=n).reshape(n, H * C)
    return out + bias


def kernel(x, edge_index, edge_attr, batch, W1l, b1l, W1r, b1r, We1, att1, bias1,
           W2l, b2l, W2r, b2r, We2, att2, bias2, fc1_w, fc1_b, fc2_w, fc2_b):
    src, dst = edge_index[0], edge_index[1]
    h = _gatv2(x, src, dst, edge_attr, W1l, b1l, W1r, b1r, We1, att1, bias1, 3, 3)
    h = jax.nn.relu(h)
    h = _gatv2(h, src, dst, edge_attr, W2l, b2l, W2r, b2r, We2, att2, bias2, 4, 5)
    h = jax.nn.relu(h)
    g = jax.ops.segment_max(h, batch, num_segments=NUM_GRAPHS)
    g = jnp.where(jnp.isfinite(g), g, 0.0)
    g = jax.nn.relu(g @ fc1_w + fc1_b)
    return g @ fc2_w + fc2_b


# speedup vs baseline: 27.3485x; 27.3485x over previous
import functools
import numpy as np
import jax
import jax.numpy as jnp
from jax import lax
from jax.experimental import pallas as pl
from jax.experimental.pallas import tpu as pltpu
from jax.experimental.pallas import tpu_sc as plsc

N = 100000
E = 1600000
G = 64
NC, NS = 2, 16
NW = NC * NS
EW = E // NW
KC = 128
NFULL = EW // KC
KT = EW - NFULL * KC
NSTRIPE = N // NS

_HI = jax.lax.Precision.HIGHEST


def _sc_gather(table, idx):
    mesh = plsc.VectorSubcoreMesh(core_axis_name="c", subcore_axis_name="s")

    @functools.partial(
        pl.kernel,
        mesh=mesh,
        out_type=jax.ShapeDtypeStruct((E, 16), jnp.float32),
        compiler_params=pltpu.CompilerParams(use_tc_tiling_on_sc=False),
        scratch_types=[
            pltpu.VMEM((KC,), jnp.int32),
            pltpu.VMEM((KC, 16), jnp.float32),
            pltpu.VMEM((KT,), jnp.int32),
            pltpu.VMEM((KT, 16), jnp.float32),
            pltpu.SemaphoreType.DMA,
        ],
    )
    def k(table_hbm, idx_hbm, out_hbm, idx_v, rows_v, idx_t, rows_t, sem):
        wid = lax.axis_index("s") * NC + lax.axis_index("c")
        base = wid * EW

        def body(i, carry):
            off = base + i * KC
            pltpu.sync_copy(idx_hbm.at[pl.ds(off, KC)], idx_v)
            pltpu.async_copy(table_hbm.at[idx_v], rows_v, sem).wait()
            pltpu.sync_copy(rows_v, out_hbm.at[pl.ds(off, KC)])
            return carry

        lax.fori_loop(0, NFULL, body, 0)
        off = base + NFULL * KC
        pltpu.sync_copy(idx_hbm.at[pl.ds(off, KT)], idx_t)
        pltpu.async_copy(table_hbm.at[idx_t], rows_t, sem).wait()
        pltpu.sync_copy(rows_t, out_hbm.at[pl.ds(off, KT)])

    return k(table, idx)


def _sc_scatter(vals, dst, zeros):
    mesh = plsc.VectorSubcoreMesh(core_axis_name="c", subcore_axis_name="s")

    @functools.partial(
        pl.kernel,
        mesh=mesh,
        out_type=jax.ShapeDtypeStruct((NC * N, 16), jnp.float32),
        compiler_params=pltpu.CompilerParams(use_tc_tiling_on_sc=False),
        scratch_types=[
            pltpu.VMEM((KC,), jnp.int32),
            pltpu.VMEM((KC, 16), jnp.float32),
            pltpu.VMEM((KT,), jnp.int32),
            pltpu.VMEM((KT, 16), jnp.float32),
            pltpu.VMEM_SHARED((N, 16), jnp.float32),
        ],
    )
    def k(vals_hbm, dst_hbm, zeros_hbm, out_hbm, idx_v, val_v, idx_t, val_t, acc_sh):
        cid = lax.axis_index("c")
        sid = lax.axis_index("s")
        wid = sid * NC + cid
        r0 = sid * NSTRIPE
        pltpu.sync_copy(zeros_hbm.at[pl.ds(r0, NSTRIPE)], acc_sh.at[pl.ds(r0, NSTRIPE)])
        plsc.subcore_barrier()

        base = wid * EW

        def body(i, carry):
            off = base + i * KC
            pltpu.sync_copy(dst_hbm.at[pl.ds(off, KC)], idx_v)
            pltpu.sync_copy(vals_hbm.at[pl.ds(off, KC)], val_v)
            pltpu.sync_copy(val_v, acc_sh.at[idx_v], add=True)
            return carry

        lax.fori_loop(0, NFULL, body, 0)
        off = base + NFULL * KC
        pltpu.sync_copy(dst_hbm.at[pl.ds(off, KT)], idx_t)
        pltpu.sync_copy(vals_hbm.at[pl.ds(off, KT)], val_t)
        pltpu.sync_copy(val_t, acc_sh.at[idx_t], add=True)

        plsc.subcore_barrier()
        pltpu.sync_copy(acc_sh.at[pl.ds(r0, NSTRIPE)],
                        out_hbm.at[pl.ds(cid * N + r0, NSTRIPE)])

    return k(vals, dst, zeros)


_BN = 2000
_BE = 8000
_BP = 1000


def _prep_nodes(xin, Wl, bl, Wr, br):
    n, din = xin.shape
    nb = n // _BN

    def body(x_ref, wl_ref, bl_ref, wr_ref, br_ref, xl_ref, xr_ref, ml_ref, mr_ref):
        i = pl.program_id(0)
        xb = x_ref[...]
        xl = jnp.dot(xb, wl_ref[...]) + bl_ref[...]
        xr = jnp.dot(xb, wr_ref[...]) + br_ref[...]
        xl_ref[...] = xl
        xr_ref[...] = xr

        @pl.when(i == 0)
        def _():
            ml_ref[...] = jnp.zeros((1, 1), jnp.float32)
            mr_ref[...] = jnp.zeros((1, 1), jnp.float32)

        ml_ref[...] = jnp.maximum(ml_ref[...], jnp.max(jnp.abs(xl)).reshape(1, 1))
        mr_ref[...] = jnp.maximum(mr_ref[...], jnp.max(jnp.abs(xr)).reshape(1, 1))

    return pl.pallas_call(
        body,
        grid=(nb,),
        in_specs=[
            pl.BlockSpec((_BN, din), lambda i: (i, 0)),
            pl.BlockSpec((din, 16), lambda i: (0, 0)),
            pl.BlockSpec((1, 16), lambda i: (0, 0)),
            pl.BlockSpec((din, 16), lambda i: (0, 0)),
            pl.BlockSpec((1, 16), lambda i: (0, 0)),
        ],
        out_specs=[
            pl.BlockSpec((_BN, 16), lambda i: (i, 0)),
            pl.BlockSpec((_BN, 16), lambda i: (i, 0)),
            pl.BlockSpec((1, 1), lambda i: (0, 0)),
            pl.BlockSpec((1, 1), lambda i: (0, 0)),
        ],
        out_shape=[
            jax.ShapeDtypeStruct((n, 16), jnp.float32),
            jax.ShapeDtypeStruct((n, 16), jnp.float32),
            jax.ShapeDtypeStruct((1, 1), jnp.float32),
            jax.ShapeDtypeStruct((1, 1), jnp.float32),
        ],
    )(xin, Wl, bl, Wr, br)


def _prep_edges(edge_attr, We):
    nb = E // _BE

    def body(ea_ref, we_ref, ef_ref, me_ref):
        i = pl.program_id(0)
        ef = jnp.dot(ea_ref[...], we_ref[...])
        ef_ref[...] = ef

        @pl.when(i == 0)
        def _():
            me_ref[...] = jnp.zeros((1, 1), jnp.float32)

        me_ref[...] = jnp.maximum(me_ref[...], jnp.max(jnp.abs(ef)).reshape(1, 1))

    return pl.pallas_call(
        body,
        grid=(nb,),
        in_specs=[
            pl.BlockSpec((_BE, 3), lambda i: (i, 0)),
            pl.BlockSpec((3, 16), lambda i: (0, 0)),
        ],
        out_specs=[
            pl.BlockSpec((_BE, 16), lambda i: (i, 0)),
            pl.BlockSpec((1, 1), lambda i: (0, 0)),
        ],
        out_shape=[
            jax.ShapeDtypeStruct((E, 16), jnp.float32),
            jax.ShapeDtypeStruct((1, 1), jnp.float32),
        ],
    )(edge_attr, We)


def _edge_compute(xlg, xrg, ef, A, cb, H, C):
    F = H * C
    nb = E // _BE
    Bm = np.zeros((16, 16), np.float32)
    Sm = np.zeros((16, 16), np.float32)
    for h in range(H):
        for c in range(C):
            Bm[h, h * C + c] = 1.0
        Sm[h, F + h] = 1.0

    def body(xl_ref, xr_ref, ef_ref, a_ref, cb_ref, bm_ref, sm_ref, out_ref):
        xl = xl_ref[...]
        m = xl + xr_ref[...] + ef_ref[...]
        m = jnp.where(m > 0, m, 0.2 * m)
        logits = jnp.dot(m, a_ref[...], precision=_HI)
        ex = jnp.exp(logits - cb_ref[...])
        out_ref[...] = (xl * jnp.dot(ex, bm_ref[...], precision=_HI)
                        + jnp.dot(ex, sm_ref[...], precision=_HI))

    return pl.pallas_call(
        body,
        grid=(nb,),
        in_specs=[
            pl.BlockSpec((_BE, 16), lambda i: (i, 0)),
            pl.BlockSpec((_BE, 16), lambda i: (i, 0)),
            pl.BlockSpec((_BE, 16), lambda i: (i, 0)),
            pl.BlockSpec((16, 16), lambda i: (0, 0)),
            pl.BlockSpec((1, 1), lambda i: (0, 0)),
            pl.BlockSpec((16, 16), lambda i: (0, 0)),
            pl.BlockSpec((16, 16), lambda i: (0, 0)),
        ],
        out_specs=pl.BlockSpec((_BE, 16), lambda i: (i, 0)),
        out_shape=jax.ShapeDtypeStruct((E, 16), jnp.float32),
    )(xlg, xrg, ef, A, cb, jnp.asarray(Bm), jnp.asarray(Sm))


def _combine(part, bias, H, C):
    F = H * C
    nb = N // _BN
    Dm = np.zeros((16, 16), np.float32)
    msk = np.zeros((1, 16), np.float32)
    for h in range(H):
        for c in range(C):
            Dm[F + h, h * C + c] = 1.0
            msk[0, h * C + c] = 1.0

    def body(p_ref, b_ref, dm_ref, msk_ref, out_ref):
        acc = p_ref[0] + p_ref[1]
        den = jnp.dot(acc, dm_ref[...], precision=_HI) + 1e-16
        outv = jnp.where(msk_ref[...] > 0, acc / den + b_ref[...], 0.0)
        out_ref[...] = jnp.maximum(outv, 0.0)

    return pl.pallas_call(
        body,
        grid=(nb,),
        in_specs=[
            pl.BlockSpec((2, _BN, 16), lambda i: (0, i, 0)),
            pl.BlockSpec((1, 16), lambda i: (0, 0)),
            pl.BlockSpec((16, 16), lambda i: (0, 0)),
            pl.BlockSpec((1, 16), lambda i: (0, 0)),
        ],
        out_specs=pl.BlockSpec((_BN, 16), lambda i: (i, 0)),
        out_shape=jax.ShapeDtypeStruct((N, 16), jnp.float32),
    )(part, bias, jnp.asarray(Dm), jnp.asarray(msk))


def _pool_mlp(h2a, h2b, onehot, fc1, fc1b, fc2, fc2b):
    nb = N // _BP

    def body(ha_ref, hb_ref, oh_ref, fc1_ref, fc1b_ref, fc2_ref, fc2b_ref,
             out_ref, ga, gb):
        i = pl.program_id(0)

        @pl.when(i == 0)
        def _():
            ga[...] = jnp.zeros((G, 16), jnp.float32)
            gb[...] = jnp.zeros((G, 16), jnp.float32)

        ha = ha_ref[...]
        hb = hb_ref[...]
        for g in range(G):
            w = oh_ref[:, g:g + 1]
            ga[g, :] = jnp.maximum(ga[g, :], jnp.max(ha * w, axis=0))
            gb[g, :] = jnp.maximum(gb[g, :], jnp.max(hb * w, axis=0))

        @pl.when(i == nb - 1)
        def _():
            gcat = jnp.concatenate(
                [ga[:, :10], gb[:, :10], jnp.zeros((G, 12), jnp.float32)], axis=1)
            z = jnp.dot(gcat, fc1_ref[...]) + fc1b_ref[...]
            z = jnp.maximum(z, 0.0)
            out_ref[...] = jnp.dot(z, fc2_ref[...]) + fc2b_ref[...]

    return pl.pallas_call(
        body,
        grid=(nb,),
        in_specs=[
            pl.BlockSpec((_BP, 16), lambda i: (i, 0)),
            pl.BlockSpec((_BP, 16), lambda i: (i, 0)),
            pl.BlockSpec((_BP, G), lambda i: (i, 0)),
            pl.BlockSpec((32, 16), lambda i: (0, 0)),
            pl.BlockSpec((1, 16), lambda i: (0, 0)),
            pl.BlockSpec((16, 1), lambda i: (0, 0)),
            pl.BlockSpec((1, 1), lambda i: (0, 0)),
        ],
        out_specs=pl.BlockSpec((G, 1), lambda i: (0, 0)),
        out_shape=jax.ShapeDtypeStruct((G, 1), jnp.float32),
        scratch_shapes=[
            pltpu.VMEM((G, 16), jnp.float32),
            pltpu.VMEM((G, 16), jnp.float32),
        ],
    )(h2a, h2b, onehot, fc1, fc1b, fc2, fc2b)


def _pad2(w, rows, cols):
    return jnp.pad(w, ((0, rows - w.shape[0]), (0, cols - w.shape[1])))


def _pad_row(b, cols):
    return jnp.pad(b, (0, cols - b.shape[0])).reshape(1, cols)


def _att_matrix(att_hc, H, C):
    cols = jnp.asarray([h * C + c for h in range(H) for c in range(C)])
    heads = jnp.asarray([h for h in range(H) for _ in range(C)])
    return jnp.zeros((16, 16), jnp.float32).at[cols, heads].set(att_hc.reshape(-1))


def kernel(x, edge_index, edge_attr, batch, W1l, b1l, W1r, b1r, We1, att1, bias1,
           W2l, b2l, W2r, b2r, We2, att2, bias2, fc1_w, fc1_b, fc2_w, fc2_b):
    src = edge_index[0]
    dst = edge_index[1]
    zeros16 = jnp.zeros((N, 16), jnp.float32)
    onehot = (batch[:, None] == jnp.arange(G, dtype=jnp.int32)[None, :]
              ).astype(jnp.float32)

    def layer_pass(xin, Wl, bl, Wr, br, We, att_hc, bias_f, H, C):
        din = xin.shape[1]
        xl, xr, ml, mr = _prep_nodes(xin, _pad2(Wl, din, 16), _pad_row(bl, 16),
                                     _pad2(Wr, din, 16), _pad_row(br, 16))
        ef, me = _prep_edges(edge_attr, _pad2(We, 3, 16))
        A = _att_matrix(att_hc, H, C)
        rowsum = jnp.max(jnp.sum(jnp.abs(att_hc), axis=1))
        cb = (ml + mr + me) * rowsum
        xlg = _sc_gather(xl, src)
        xrg = _sc_gather(xr, dst)
        vals = _edge_compute(xlg, xrg, ef, A, cb, H, C)
        part = _sc_scatter(vals, dst, zeros16).reshape(NC, N, 16)
        return _combine(part, _pad_row(bias_f, 16), H, C)

    h1 = layer_pass(x, W1l, b1l, W1r, b1r, We1, att1, bias1, 3, 3)
    h2a = layer_pass(h1, W2l[:, :10], b2l[:10], W2r[:, :10], b2r[:10],
                     We2[:, :10], att2[0:2], bias2[:10], 2, 5)
    h2b = layer_pass(h1, W2l[:, 10:], b2l[10:], W2r[:, 10:], b2r[10:],
                     We2[:, 10:], att2[2:4], bias2[10:], 2, 5)

    return _pool_mlp(h2a, h2b, onehot,
                     _pad2(fc1_w, 32, 16), _pad_row(fc1_b, 16),
                     _pad2(fc2_w, 16, 1), fc2_b.reshape(1, 1))


# speedup vs baseline: 29.3340x; 1.0726x over previous
import functools
import numpy as np
import jax
import jax.numpy as jnp
from jax import lax
from jax.experimental import pallas as pl
from jax.experimental.pallas import tpu as pltpu
from jax.experimental.pallas import tpu_sc as plsc

N = 100000
E = 1600000
G = 64
NC, NS = 2, 16
NW = NC * NS
KC = 128
R = E // KC
KB = 8
RFULL, RREM = divmod(R, NW)
NOUT = RFULL // KB
TL_BIG = RFULL + 1 - NOUT * KB
TL_SMALL = RFULL - NOUT * KB
NSTRIPE = N // NS

_HI = jax.lax.Precision.HIGHEST


def _sc_gather(table, idx):
    mesh = plsc.VectorSubcoreMesh(core_axis_name="c", subcore_axis_name="s")

    @functools.partial(
        pl.kernel,
        mesh=mesh,
        out_type=jax.ShapeDtypeStruct((E, 16), jnp.float32),
        compiler_params=pltpu.CompilerParams(use_tc_tiling_on_sc=False),
        scratch_types=[
            pltpu.VMEM((KB * KC,), jnp.int32),
            pltpu.VMEM((KB * KC, 16), jnp.float32),
            pltpu.SemaphoreType.DMA,
        ],
    )
    def k(table_hbm, idx_hbm, out_hbm, idx_v, rows_v, sem):
        wid = lax.axis_index("s") * NC + lax.axis_index("c")
        r0 = RFULL * wid + jnp.minimum(wid, RREM)

        def run(e0, nb):
            pltpu.sync_copy(idx_hbm.at[pl.ds(e0, nb * KC)], idx_v.at[pl.ds(0, nb * KC)])
            cps = [
                pltpu.async_copy(table_hbm.at[idx_v.at[pl.ds(b * KC, KC)]],
                                 rows_v.at[pl.ds(b * KC, KC)], sem)
                for b in range(nb)
            ]
            for c in cps:
                c.wait()
            pltpu.sync_copy(rows_v.at[pl.ds(0, nb * KC)], out_hbm.at[pl.ds(e0, nb * KC)])

        def body(o, carry):
            run((r0 + o * KB) * KC, KB)
            return carry

        lax.fori_loop(0, NOUT, body, 0)
        e0 = (r0 + NOUT * KB) * KC

        @pl.when(wid < RREM)
        def _():
            run(e0, TL_BIG)

        @pl.when(wid >= RREM)
        def _():
            run(e0, TL_SMALL)

    return k(table, idx)


def _sc_scatter(vals, dst2, zeros):
    mesh = plsc.VectorSubcoreMesh(core_axis_name="c", subcore_axis_name="s")

    @functools.partial(
        pl.kernel,
        mesh=mesh,
        out_type=jax.ShapeDtypeStruct((NC * N, 16), jnp.float32),
        compiler_params=pltpu.CompilerParams(use_tc_tiling_on_sc=False),
        scratch_types=[
            pltpu.VMEM((KB, KC), jnp.int32),
            pltpu.VMEM((KB * KC, 16), jnp.float32),
            pltpu.VMEM_SHARED((N, 16), jnp.float32),
            pltpu.SemaphoreType.DMA,
        ],
    )
    def k(vals_hbm, dst_hbm, zeros_hbm, out_hbm, idx_v, val_v, acc_sh, sem):
        cid = lax.axis_index("c")
        sid = lax.axis_index("s")
        wid = sid * NC + cid
        s0 = sid * NSTRIPE
        pltpu.sync_copy(zeros_hbm.at[pl.ds(s0, NSTRIPE)], acc_sh.at[pl.ds(s0, NSTRIPE)])
        plsc.subcore_barrier()

        r0 = RFULL * wid + jnp.minimum(wid, RREM)

        def run(row, nb):
            pltpu.sync_copy(dst_hbm.at[pl.ds(row, nb)], idx_v.at[pl.ds(0, nb)])
            pltpu.sync_copy(vals_hbm.at[pl.ds(row * KC, nb * KC)],
                            val_v.at[pl.ds(0, nb * KC)])
            cps = [
                pltpu.async_copy(val_v.at[pl.ds(b * KC, KC)],
                                 acc_sh.at[idx_v.at[b]], sem, add=True)
                for b in range(nb)
            ]
            for c in cps:
                c.wait()

        def body(o, carry):
            run(r0 + o * KB, KB)
            return carry

        lax.fori_loop(0, NOUT, body, 0)
        row = r0 + NOUT * KB

        @pl.when(wid < RREM)
        def _():
            run(row, TL_BIG)

        @pl.when(wid >= RREM)
        def _():
            run(row, TL_SMALL)

        plsc.subcore_barrier()
        pltpu.sync_copy(acc_sh.at[pl.ds(s0, NSTRIPE)],
                        out_hbm.at[pl.ds(cid * N + s0, NSTRIPE)])

    return k(vals, dst2, zeros)


_BN = 2000
_BE = 8000
_BP = 1000


def _prep_nodes(xin, Wl, bl, Wr, br):
    n, din = xin.shape
    nb = n // _BN

    def body(x_ref, wl_ref, bl_ref, wr_ref, br_ref, xl_ref, xr_ref, ml_ref, mr_ref):
        i = pl.program_id(0)
        xb = x_ref[...]
        xl = jnp.dot(xb, wl_ref[...]) + bl_ref[...]
        xr = jnp.dot(xb, wr_ref[...]) + br_ref[...]
        xl_ref[...] = xl
        xr_ref[...] = xr

        @pl.when(i == 0)
        def _():
            ml_ref[...] = jnp.zeros((1, 1), jnp.float32)
            mr_ref[...] = jnp.zeros((1, 1), jnp.float32)

        ml_ref[...] = jnp.maximum(ml_ref[...], jnp.max(jnp.abs(xl)).reshape(1, 1))
        mr_ref[...] = jnp.maximum(mr_ref[...], jnp.max(jnp.abs(xr)).reshape(1, 1))

    return pl.pallas_call(
        body,
        grid=(nb,),
        in_specs=[
            pl.BlockSpec((_BN, din), lambda i: (i, 0)),
            pl.BlockSpec((din, 16), lambda i: (0, 0)),
            pl.BlockSpec((1, 16), lambda i: (0, 0)),
            pl.BlockSpec((din, 16), lambda i: (0, 0)),
            pl.BlockSpec((1, 16), lambda i: (0, 0)),
        ],
        out_specs=[
            pl.BlockSpec((_BN, 16), lambda i: (i, 0)),
            pl.BlockSpec((_BN, 16), lambda i: (i, 0)),
            pl.BlockSpec((1, 1), lambda i: (0, 0)),
            pl.BlockSpec((1, 1), lambda i: (0, 0)),
        ],
        out_shape=[
            jax.ShapeDtypeStruct((n, 16), jnp.float32),
            jax.ShapeDtypeStruct((n, 16), jnp.float32),
            jax.ShapeDtypeStruct((1, 1), jnp.float32),
            jax.ShapeDtypeStruct((1, 1), jnp.float32),
        ],
    )(xin, Wl, bl, Wr, br)


def _prep_edges(edge_attr, We):
    nb = E // _BE

    def body(ea_ref, we_ref, ef_ref, me_ref):
        i = pl.program_id(0)
        ef = jnp.dot(ea_ref[...], we_ref[...])
        ef_ref[...] = ef

        @pl.when(i == 0)
        def _():
            me_ref[...] = jnp.zeros((1, 1), jnp.float32)

        me_ref[...] = jnp.maximum(me_ref[...], jnp.max(jnp.abs(ef)).reshape(1, 1))

    return pl.pallas_call(
        body,
        grid=(nb,),
        in_specs=[
            pl.BlockSpec((_BE, 3), lambda i: (i, 0)),
            pl.BlockSpec((3, 16), lambda i: (0, 0)),
        ],
        out_specs=[
            pl.BlockSpec((_BE, 16), lambda i: (i, 0)),
            pl.BlockSpec((1, 1), lambda i: (0, 0)),
        ],
        out_shape=[
            jax.ShapeDtypeStruct((E, 16), jnp.float32),
            jax.ShapeDtypeStruct((1, 1), jnp.float32),
        ],
    )(edge_attr, We)


def _edge_compute(xlg, xrg, ef, A, cb, H, C):
    F = H * C
    nb = E // _BE
    Bm = np.zeros((16, 16), np.float32)
    Sm = np.zeros((16, 16), np.float32)
    for h in range(H):
        for c in range(C):
            Bm[h, h * C + c] = 1.0
        Sm[h, F + h] = 1.0

    def body(xl_ref, xr_ref, ef_ref, a_ref, cb_ref, bm_ref, sm_ref, out_ref):
        xl = xl_ref[...]
        m = xl + xr_ref[...] + ef_ref[...]
        m = jnp.where(m > 0, m, 0.2 * m)
        logits = jnp.dot(m, a_ref[...], precision=_HI)
        ex = jnp.exp(logits - cb_ref[...])
        out_ref[...] = (xl * jnp.dot(ex, bm_ref[...], precision=_HI)
                        + jnp.dot(ex, sm_ref[...], precision=_HI))

    return pl.pallas_call(
        body,
        grid=(nb,),
        in_specs=[
            pl.BlockSpec((_BE, 16), lambda i: (i, 0)),
            pl.BlockSpec((_BE, 16), lambda i: (i, 0)),
            pl.BlockSpec((_BE, 16), lambda i: (i, 0)),
            pl.BlockSpec((16, 16), lambda i: (0, 0)),
            pl.BlockSpec((1, 1), lambda i: (0, 0)),
            pl.BlockSpec((16, 16), lambda i: (0, 0)),
            pl.BlockSpec((16, 16), lambda i: (0, 0)),
        ],
        out_specs=pl.BlockSpec((_BE, 16), lambda i: (i, 0)),
        out_shape=jax.ShapeDtypeStruct((E, 16), jnp.float32),
    )(xlg, xrg, ef, A, cb, jnp.asarray(Bm), jnp.asarray(Sm))


def _combine(part, bias, H, C):
    F = H * C
    nb = N // _BN
    Dm = np.zeros((16, 16), np.float32)
    msk = np.zeros((1, 16), np.float32)
    for h in range(H):
        for c in range(C):
            Dm[F + h, h * C + c] = 1.0
            msk[0, h * C + c] = 1.0

    def body(p_ref, b_ref, dm_ref, msk_ref, out_ref):
        acc = p_ref[0] + p_ref[1]
        den = jnp.dot(acc, dm_ref[...], precision=_HI) + 1e-16
        outv = jnp.where(msk_ref[...] > 0, acc / den + b_ref[...], 0.0)
        out_ref[...] = jnp.maximum(outv, 0.0)

    return pl.pallas_call(
        body,
        grid=(nb,),
        in_specs=[
            pl.BlockSpec((2, _BN, 16), lambda i: (0, i, 0)),
            pl.BlockSpec((1, 16), lambda i: (0, 0)),
            pl.BlockSpec((16, 16), lambda i: (0, 0)),
            pl.BlockSpec((1, 16), lambda i: (0, 0)),
        ],
        out_specs=pl.BlockSpec((_BN, 16), lambda i: (i, 0)),
        out_shape=jax.ShapeDtypeStruct((N, 16), jnp.float32),
    )(part, bias, jnp.asarray(Dm), jnp.asarray(msk))


def _pool_mlp(h2a, h2b, onehot, fc1, fc1b, fc2, fc2b):
    nb = N // _BP

    def body(ha_ref, hb_ref, oh_ref, fc1_ref, fc1b_ref, fc2_ref, fc2b_ref,
             out_ref, ga, gb):
        i = pl.program_id(0)

        @pl.when(i == 0)
        def _():
            ga[...] = jnp.zeros((G, 16), jnp.float32)
            gb[...] = jnp.zeros((G, 16), jnp.float32)

        ha = ha_ref[...]
        hb = hb_ref[...]
        for g in range(G):
            w = oh_ref[:, g:g + 1]
            ga[g, :] = jnp.maximum(ga[g, :], jnp.max(ha * w, axis=0))
            gb[g, :] = jnp.maximum(gb[g, :], jnp.max(hb * w, axis=0))

        @pl.when(i == nb - 1)
        def _():
            gcat = jnp.concatenate(
                [ga[:, :10], gb[:, :10], jnp.zeros((G, 12), jnp.float32)], axis=1)
            z = jnp.dot(gcat, fc1_ref[...]) + fc1b_ref[...]
            z = jnp.maximum(z, 0.0)
            out_ref[...] = jnp.dot(z, fc2_ref[...]) + fc2b_ref[...]

    return pl.pallas_call(
        body,
        grid=(nb,),
        in_specs=[
            pl.BlockSpec((_BP, 16), lambda i: (i, 0)),
            pl.BlockSpec((_BP, 16), lambda i: (i, 0)),
            pl.BlockSpec((_BP, G), lambda i: (i, 0)),
            pl.BlockSpec((32, 16), lambda i: (0, 0)),
            pl.BlockSpec((1, 16), lambda i: (0, 0)),
            pl.BlockSpec((16, 1), lambda i: (0, 0)),
            pl.BlockSpec((1, 1), lambda i: (0, 0)),
        ],
        out_specs=pl.BlockSpec((G, 1), lambda i: (0, 0)),
        out_shape=jax.ShapeDtypeStruct((G, 1), jnp.float32),
        scratch_shapes=[
            pltpu.VMEM((G, 16), jnp.float32),
            pltpu.VMEM((G, 16), jnp.float32),
        ],
    )(h2a, h2b, onehot, fc1, fc1b, fc2, fc2b)


def _pad2(w, rows, cols):
    return jnp.pad(w, ((0, rows - w.shape[0]), (0, cols - w.shape[1])))


def _pad_row(b, cols):
    return jnp.pad(b, (0, cols - b.shape[0])).reshape(1, cols)


def _att_matrix(att_hc, H, C):
    cols = jnp.asarray([h * C + c for h in range(H) for c in range(C)])
    heads = jnp.asarray([h for h in range(H) for _ in range(C)])
    return jnp.zeros((16, 16), jnp.float32).at[cols, heads].set(att_hc.reshape(-1))


def kernel(x, edge_index, edge_attr, batch, W1l, b1l, W1r, b1r, We1, att1, bias1,
           W2l, b2l, W2r, b2r, We2, att2, bias2, fc1_w, fc1_b, fc2_w, fc2_b):
    src = edge_index[0]
    dst = edge_index[1]
    dst2 = dst.reshape(R, KC)
    zeros16 = jnp.zeros((N, 16), jnp.float32)
    onehot = (batch[:, None] == jnp.arange(G, dtype=jnp.int32)[None, :]
              ).astype(jnp.float32)

    def layer_pass(xin, Wl, bl, Wr, br, We, att_hc, bias_f, H, C):
        din = xin.shape[1]
        xl, xr, ml, mr = _prep_nodes(xin, _pad2(Wl, din, 16), _pad_row(bl, 16),
                                     _pad2(Wr, din, 16), _pad_row(br, 16))
        ef, me = _prep_edges(edge_attr, _pad2(We, 3, 16))
        A = _att_matrix(att_hc, H, C)
        rowsum = jnp.max(jnp.sum(jnp.abs(att_hc), axis=1))
        cb = (ml + mr + me) * rowsum
        xlg = _sc_gather(xl, src)
        xrg = _sc_gather(xr, dst)
        vals = _edge_compute(xlg, xrg, ef, A, cb, H, C)
        part = _sc_scatter(vals, dst2, zeros16).reshape(NC, N, 16)
        return _combine(part, _pad_row(bias_f, 16), H, C)

    h1 = layer_pass(x, W1l, b1l, W1r, b1r, We1, att1, bias1, 3, 3)
    h2a = layer_pass(h1, W2l[:, :10], b2l[:10], W2r[:, :10], b2r[:10],
                     We2[:, :10], att2[0:2], bias2[:10], 2, 5)
    h2b = layer_pass(h1, W2l[:, 10:], b2l[10:], W2r[:, 10:], b2r[10:],
                     We2[:, 10:], att2[2:4], bias2[10:], 2, 5)

    return _pool_mlp(h2a, h2b, onehot,
                     _pad2(fc1_w, 32, 16), _pad_row(fc1_b, 16),
                     _pad2(fc2_w, 16, 1), fc2_b.reshape(1, 1))


# speedup vs baseline: 67.0348x; 2.2852x over previous
import functools
import numpy as np
import jax
import jax.numpy as jnp
from jax import lax
from jax.experimental import pallas as pl
from jax.experimental.pallas import tpu as pltpu
from jax.experimental.pallas import tpu_sc as plsc

N = 100000
E = 1600000
G = 64
NC, NS = 2, 16
NW = NC * NS
KC = 128
R = E // KC
KB = 8
RFULL, RREM = divmod(R, NW)
NOUT = RFULL // KB
TL_BIG = RFULL + 1 - NOUT * KB
TL_SMALL = RFULL - NOUT * KB
NSTRIPE = N // NS

_HI = jax.lax.Precision.HIGHEST


def _sc_gather(table, idx):
    mesh = plsc.VectorSubcoreMesh(core_axis_name="c", subcore_axis_name="s")

    @functools.partial(
        pl.kernel,
        mesh=mesh,
        out_type=jax.ShapeDtypeStruct((E, 16), jnp.float32),
        compiler_params=pltpu.CompilerParams(use_tc_tiling_on_sc=False),
        scratch_types=[
            pltpu.VMEM((KB * KC,), jnp.int32),
            pltpu.VMEM((KB * KC, 16), jnp.float32),
            pltpu.SemaphoreType.DMA,
        ],
    )
    def k(table_hbm, idx_hbm, out_hbm, idx_v, rows_v, sem):
        wid = lax.axis_index("s") * NC + lax.axis_index("c")
        r0 = RFULL * wid + jnp.minimum(wid, RREM)

        def run(e0, nb):
            pltpu.sync_copy(idx_hbm.at[pl.ds(e0, nb * KC)], idx_v.at[pl.ds(0, nb * KC)])
            cps = [
                pltpu.async_copy(table_hbm.at[idx_v.at[pl.ds(b * KC, KC)]],
                                 rows_v.at[pl.ds(b * KC, KC)], sem)
                for b in range(nb)
            ]
            for c in cps:
                c.wait()
            pltpu.sync_copy(rows_v.at[pl.ds(0, nb * KC)], out_hbm.at[pl.ds(e0, nb * KC)])

        def body(o, carry):
            run((r0 + o * KB) * KC, KB)
            return carry

        lax.fori_loop(0, NOUT, body, 0)
        e0 = (r0 + NOUT * KB) * KC

        @pl.when(wid < RREM)
        def _():
            run(e0, TL_BIG)

        @pl.when(wid >= RREM)
        def _():
            run(e0, TL_SMALL)

    return k(table, idx)


def _sc_scatter(vals, dst2, zeros):
    mesh = plsc.VectorSubcoreMesh(core_axis_name="c", subcore_axis_name="s")

    @functools.partial(
        pl.kernel,
        mesh=mesh,
        out_type=jax.ShapeDtypeStruct((NC * N, 16), jnp.float32),
        compiler_params=pltpu.CompilerParams(use_tc_tiling_on_sc=False),
        scratch_types=[
            pltpu.VMEM((KB, KC), jnp.int32),
            pltpu.VMEM((KB * KC, 16), jnp.float32),
            pltpu.VMEM_SHARED((N, 16), jnp.float32),
            pltpu.SemaphoreType.DMA,
        ],
    )
    def k(vals_hbm, dst_hbm, zeros_hbm, out_hbm, idx_v, val_v, acc_sh, sem):
        cid = lax.axis_index("c")
        sid = lax.axis_index("s")
        wid = sid * NC + cid
        s0 = sid * NSTRIPE
        pltpu.sync_copy(zeros_hbm.at[pl.ds(s0, NSTRIPE)], acc_sh.at[pl.ds(s0, NSTRIPE)])
        plsc.subcore_barrier()

        r0 = RFULL * wid + jnp.minimum(wid, RREM)

        def run(row, nb):
            pltpu.sync_copy(dst_hbm.at[pl.ds(row, nb)], idx_v.at[pl.ds(0, nb)])
            pltpu.sync_copy(vals_hbm.at[pl.ds(row * KC, nb * KC)],
                            val_v.at[pl.ds(0, nb * KC)])
            cps = [
                pltpu.async_copy(val_v.at[pl.ds(b * KC, KC)],
                                 acc_sh.at[idx_v.at[b]], sem, add=True)
                for b in range(nb)
            ]
            for c in cps:
                c.wait()

        def body(o, carry):
            run(r0 + o * KB, KB)
            return carry

        lax.fori_loop(0, NOUT, body, 0)
        row = r0 + NOUT * KB

        @pl.when(wid < RREM)
        def _():
            run(row, TL_BIG)

        @pl.when(wid >= RREM)
        def _():
            run(row, TL_SMALL)

        plsc.subcore_barrier()
        pltpu.sync_copy(acc_sh.at[pl.ds(s0, NSTRIPE)],
                        out_hbm.at[pl.ds(cid * N + s0, NSTRIPE)])

    return k(vals, dst2, zeros)


_BN = 2000
_BE = 8000
_BP = 1000


def _prep_nodes(xin, Wl, bl, Wr, br):
    n, din = xin.shape
    nb = n // _BN

    def body(x_ref, wl_ref, bl_ref, wr_ref, br_ref, xl_ref, xr_ref, ml_ref, mr_ref):
        i = pl.program_id(0)
        xb = x_ref[...]
        xl = jnp.dot(xb, wl_ref[...]) + bl_ref[...]
        xr = jnp.dot(xb, wr_ref[...]) + br_ref[...]
        xl_ref[...] = xl
        xr_ref[...] = xr

        @pl.when(i == 0)
        def _():
            ml_ref[...] = jnp.zeros((1, 1), jnp.float32)
            mr_ref[...] = jnp.zeros((1, 1), jnp.float32)

        ml_ref[...] = jnp.maximum(ml_ref[...], jnp.max(jnp.abs(xl)).reshape(1, 1))
        mr_ref[...] = jnp.maximum(mr_ref[...], jnp.max(jnp.abs(xr)).reshape(1, 1))

    return pl.pallas_call(
        body,
        grid=(nb,),
        in_specs=[
            pl.BlockSpec((_BN, din), lambda i: (i, 0)),
            pl.BlockSpec((din, 16), lambda i: (0, 0)),
            pl.BlockSpec((1, 16), lambda i: (0, 0)),
            pl.BlockSpec((din, 16), lambda i: (0, 0)),
            pl.BlockSpec((1, 16), lambda i: (0, 0)),
        ],
        out_specs=[
            pl.BlockSpec((_BN, 16), lambda i: (i, 0)),
            pl.BlockSpec((_BN, 16), lambda i: (i, 0)),
            pl.BlockSpec((1, 1), lambda i: (0, 0)),
            pl.BlockSpec((1, 1), lambda i: (0, 0)),
        ],
        out_shape=[
            jax.ShapeDtypeStruct((n, 16), jnp.float32),
            jax.ShapeDtypeStruct((n, 16), jnp.float32),
            jax.ShapeDtypeStruct((1, 1), jnp.float32),
            jax.ShapeDtypeStruct((1, 1), jnp.float32),
        ],
    )(xin, Wl, bl, Wr, br)


def _prep_edges(edge_attr, We):
    nb = E // _BE

    def body(ea_ref, we_ref, ef_ref, me_ref):
        i = pl.program_id(0)
        ef = jnp.dot(ea_ref[...], we_ref[...])
        ef_ref[...] = ef

        @pl.when(i == 0)
        def _():
            me_ref[...] = jnp.zeros((1, 1), jnp.float32)

        me_ref[...] = jnp.maximum(me_ref[...], jnp.max(jnp.abs(ef)).reshape(1, 1))

    return pl.pallas_call(
        body,
        grid=(nb,),
        in_specs=[
            pl.BlockSpec((_BE, 3), lambda i: (i, 0)),
            pl.BlockSpec((3, 16), lambda i: (0, 0)),
        ],
        out_specs=[
            pl.BlockSpec((_BE, 16), lambda i: (i, 0)),
            pl.BlockSpec((1, 1), lambda i: (0, 0)),
        ],
        out_shape=[
            jax.ShapeDtypeStruct((E, 16), jnp.float32),
            jax.ShapeDtypeStruct((1, 1), jnp.float32),
        ],
    )(edge_attr, We)


def _edge_compute(xlg, xrg, ef, A, cb, H, C):
    F = H * C
    E8 = E // 8
    BK = 2000
    nb = E8 // BK
    Bm = np.zeros((16, 16), np.float32)
    Sm = np.zeros((16, 16), np.float32)
    for h in range(H):
        for c in range(C):
            Bm[h, h * C + c] = 1.0
        Sm[h, F + h] = 1.0
    eye8 = np.eye(8, dtype=np.float32)
    Bm128 = np.kron(eye8, Bm)
    Sm128 = np.kron(eye8, Sm)
    A128 = jnp.kron(jnp.asarray(eye8), A)

    def body(xl_ref, xr_ref, ef_ref, a_ref, cb_ref, bm_ref, sm_ref, out_ref):
        xl = xl_ref[...]
        m = xl + xr_ref[...] + ef_ref[...]
        m = jnp.where(m > 0, m, 0.2 * m)
        logits = jnp.dot(m, a_ref[...], precision=_HI)
        ex = jnp.exp(logits - cb_ref[...])
        out_ref[...] = (xl * jnp.dot(ex, bm_ref[...], precision=_HI)
                        + jnp.dot(ex, sm_ref[...], precision=_HI))

    out = pl.pallas_call(
        body,
        grid=(nb,),
        in_specs=[
            pl.BlockSpec((BK, 128), lambda i: (i, 0)),
            pl.BlockSpec((BK, 128), lambda i: (i, 0)),
            pl.BlockSpec((BK, 128), lambda i: (i, 0)),
            pl.BlockSpec((128, 128), lambda i: (0, 0)),
            pl.BlockSpec((1, 1), lambda i: (0, 0)),
            pl.BlockSpec((128, 128), lambda i: (0, 0)),
            pl.BlockSpec((128, 128), lambda i: (0, 0)),
        ],
        out_specs=pl.BlockSpec((BK, 128), lambda i: (i, 0)),
        out_shape=jax.ShapeDtypeStruct((E8, 128), jnp.float32),
    )(xlg.reshape(E8, 128), xrg.reshape(E8, 128), ef.reshape(E8, 128),
      A128, cb, jnp.asarray(Bm128), jnp.asarray(Sm128))
    return out.reshape(E, 16)


def _combine(part, bias, H, C):
    F = H * C
    nb = N // _BN
    Dm = np.zeros((16, 16), np.float32)
    msk = np.zeros((1, 16), np.float32)
    for h in range(H):
        for c in range(C):
            Dm[F + h, h * C + c] = 1.0
            msk[0, h * C + c] = 1.0

    def body(p_ref, b_ref, dm_ref, msk_ref, out_ref):
        acc = p_ref[0] + p_ref[1]
        den = jnp.dot(acc, dm_ref[...], precision=_HI) + 1e-16
        outv = jnp.where(msk_ref[...] > 0, acc / den + b_ref[...], 0.0)
        out_ref[...] = jnp.maximum(outv, 0.0)

    return pl.pallas_call(
        body,
        grid=(nb,),
        in_specs=[
            pl.BlockSpec((2, _BN, 16), lambda i: (0, i, 0)),
            pl.BlockSpec((1, 16), lambda i: (0, 0)),
            pl.BlockSpec((16, 16), lambda i: (0, 0)),
            pl.BlockSpec((1, 16), lambda i: (0, 0)),
        ],
        out_specs=pl.BlockSpec((_BN, 16), lambda i: (i, 0)),
        out_shape=jax.ShapeDtypeStruct((N, 16), jnp.float32),
    )(part, bias, jnp.asarray(Dm), jnp.asarray(msk))


def _pool_mlp(h2a, h2b, onehot, fc1, fc1b, fc2, fc2b):
    nb = N // _BP

    def body(ha_ref, hb_ref, oh_ref, fc1_ref, fc1b_ref, fc2_ref, fc2b_ref,
             out_ref, ga, gb):
        i = pl.program_id(0)

        @pl.when(i == 0)
        def _():
            ga[...] = jnp.zeros((G, 16), jnp.float32)
            gb[...] = jnp.zeros((G, 16), jnp.float32)

        ha = ha_ref[...]
        hb = hb_ref[...]
        rows_a = []
        rows_b = []
        for g in range(G):
            w = oh_ref[:, g:g + 1]
            rows_a.append(jnp.max(ha * w, axis=0, keepdims=True))
            rows_b.append(jnp.max(hb * w, axis=0, keepdims=True))
        ga[...] = jnp.maximum(ga[...], jnp.concatenate(rows_a, axis=0))
        gb[...] = jnp.maximum(gb[...], jnp.concatenate(rows_b, axis=0))

        @pl.when(i == nb - 1)
        def _():
            gcat = jnp.concatenate(
                [ga[:, :10], gb[:, :10], jnp.zeros((G, 12), jnp.float32)], axis=1)
            z = jnp.dot(gcat, fc1_ref[...]) + fc1b_ref[...]
            z = jnp.maximum(z, 0.0)
            out_ref[...] = jnp.dot(z, fc2_ref[...]) + fc2b_ref[...]

    return pl.pallas_call(
        body,
        grid=(nb,),
        in_specs=[
            pl.BlockSpec((_BP, 16), lambda i: (i, 0)),
            pl.BlockSpec((_BP, 16), lambda i: (i, 0)),
            pl.BlockSpec((_BP, G), lambda i: (i, 0)),
            pl.BlockSpec((32, 16), lambda i: (0, 0)),
            pl.BlockSpec((1, 16), lambda i: (0, 0)),
            pl.BlockSpec((16, 1), lambda i: (0, 0)),
            pl.BlockSpec((1, 1), lambda i: (0, 0)),
        ],
        out_specs=pl.BlockSpec((G, 1), lambda i: (0, 0)),
        out_shape=jax.ShapeDtypeStruct((G, 1), jnp.float32),
        scratch_shapes=[
            pltpu.VMEM((G, 16), jnp.float32),
            pltpu.VMEM((G, 16), jnp.float32),
        ],
    )(h2a, h2b, onehot, fc1, fc1b, fc2, fc2b)


def _pad2(w, rows, cols):
    return jnp.pad(w, ((0, rows - w.shape[0]), (0, cols - w.shape[1])))


def _pad_row(b, cols):
    return jnp.pad(b, (0, cols - b.shape[0])).reshape(1, cols)


def _att_matrix(att_hc, H, C):
    cols = jnp.asarray([h * C + c for h in range(H) for c in range(C)])
    heads = jnp.asarray([h for h in range(H) for _ in range(C)])
    return jnp.zeros((16, 16), jnp.float32).at[cols, heads].set(att_hc.reshape(-1))


def kernel(x, edge_index, edge_attr, batch, W1l, b1l, W1r, b1r, We1, att1, bias1,
           W2l, b2l, W2r, b2r, We2, att2, bias2, fc1_w, fc1_b, fc2_w, fc2_b):
    src = edge_index[0]
    dst = edge_index[1]
    dst2 = dst.reshape(R, KC)
    zeros16 = jnp.zeros((N, 16), jnp.float32)
    onehot = (batch[:, None] == jnp.arange(G, dtype=jnp.int32)[None, :]
              ).astype(jnp.float32)

    def layer_pass(xin, Wl, bl, Wr, br, We, att_hc, bias_f, H, C):
        din = xin.shape[1]
        xl, xr, ml, mr = _prep_nodes(xin, _pad2(Wl, din, 16), _pad_row(bl, 16),
                                     _pad2(Wr, din, 16), _pad_row(br, 16))
        ef, me = _prep_edges(edge_attr, _pad2(We, 3, 16))
        A = _att_matrix(att_hc, H, C)
        rowsum = jnp.max(jnp.sum(jnp.abs(att_hc), axis=1))
        cb = (ml + mr + me) * rowsum
        xlg = _sc_gather(xl, src)
        xrg = _sc_gather(xr, dst)
        vals = _edge_compute(xlg, xrg, ef, A, cb, H, C)
        part = _sc_scatter(vals, dst2, zeros16).reshape(NC, N, 16)
        return _combine(part, _pad_row(bias_f, 16), H, C)

    h1 = layer_pass(x, W1l, b1l, W1r, b1r, We1, att1, bias1, 3, 3)
    h2a = layer_pass(h1, W2l[:, :10], b2l[:10], W2r[:, :10], b2r[:10],
                     We2[:, :10], att2[0:2], bias2[:10], 2, 5)
    h2b = layer_pass(h1, W2l[:, 10:], b2l[10:], W2r[:, 10:], b2r[10:],
                     We2[:, 10:], att2[2:4], bias2[10:], 2, 5)

    return _pool_mlp(h2a, h2b, onehot,
                     _pad2(fc1_w, 32, 16), _pad_row(fc1_b, 16),
                     _pad2(fc2_w, 16, 1), fc2_b.reshape(1, 1))
